# baseline (speedup 1.0000x reference)
import functools

import jax
import jax.numpy as jnp
from jax import lax
from jax.experimental import pallas as pl
from jax.experimental.pallas import tpu as pltpu
from jax.experimental.pallas import tpu_sc as plsc

L = 16
NC = 2
NS = 16
NW = NC * NS
B = 16384
D = 64
BPW = B // NW
CHUNK = 128
NCH = BPW // CHUNK


def _sc_body(feat_hbm, lab_hbm, centers_hbm, out_hbm,
             idx_v, rows_v, feat_v, acc_v, gsem, fsem):
    wid = lax.axis_index("s") * NC + lax.axis_index("c")
    base = wid * BPW

    pltpu.sync_copy(lab_hbm.at[pl.ds(wid * NCH, NCH)], idx_v)

    fcopy = pltpu.async_copy(feat_hbm.at[pl.ds(base * D, BPW * D)], feat_v, fsem)
    gcopies = []
    for j in range(NCH):
        gcopies.append(pltpu.async_copy(
            centers_hbm.at[idx_v.at[j]],
            rows_v.at[pl.ds(j * CHUNK, CHUNK)], gsem))
    fcopy.wait()
    for c in gcopies:
        c.wait()

    def row_body(i, accs):
        a0, a1, a2, a3 = accs
        f0 = feat_v[pl.ds(i * D + 0 * L, L)]
        f1 = feat_v[pl.ds(i * D + 1 * L, L)]
        f2 = feat_v[pl.ds(i * D + 2 * L, L)]
        f3 = feat_v[pl.ds(i * D + 3 * L, L)]
        c0 = rows_v[i, pl.ds(0 * L, L)]
        c1 = rows_v[i, pl.ds(1 * L, L)]
        c2 = rows_v[i, pl.ds(2 * L, L)]
        c3 = rows_v[i, pl.ds(3 * L, L)]
        d0 = f0 - c0
        d1 = f1 - c1
        d2 = f2 - c2
        d3 = f3 - c3
        return (a0 + d0 * d0, a1 + d1 * d1, a2 + d2 * d2, a3 + d3 * d3)

    zero = jnp.zeros((L,), jnp.float32)
    a0, a1, a2, a3 = lax.fori_loop(0, BPW, row_body, (zero, zero, zero, zero))
    acc_v[...] = (a0 + a1) + (a2 + a3)
    pltpu.sync_copy(acc_v, out_hbm.at[wid])


@functools.partial(
    pl.kernel,
    out_type=jax.ShapeDtypeStruct((NW, L), jnp.float32),
    mesh=plsc.VectorSubcoreMesh(core_axis_name="c", subcore_axis_name="s"),
    compiler_params=pltpu.CompilerParams(use_tc_tiling_on_sc=False),
    scratch_types=[
        pltpu.VMEM((NCH, CHUNK), jnp.int32),
        pltpu.VMEM((BPW, D), jnp.float32),
        pltpu.VMEM((BPW * D,), jnp.float32),
        pltpu.VMEM((L,), jnp.float32),
        pltpu.SemaphoreType.DMA,
        pltpu.SemaphoreType.DMA,
    ],
)
def _centerloss_partials(feat_hbm, lab_hbm, centers_hbm, out_hbm,
                         idx_v, rows_v, feat_v, acc_v, gsem, fsem):
    _sc_body(feat_hbm, lab_hbm, centers_hbm, out_hbm,
             idx_v, rows_v, feat_v, acc_v, gsem, fsem)


def kernel(feature, label, centers, batch_size):
    feat_flat = feature.reshape(B * D)
    lab2d = label.astype(jnp.int32).reshape(B // CHUNK, CHUNK)
    partials = _centerloss_partials(feat_flat, lab2d, centers)
    return jnp.sum(partials) / 2.0 / batch_size
